# baseline (speedup 1.0000x reference)
import jax
import jax.numpy as jnp
from jax.experimental import pallas as pl
from jax.experimental.pallas import tpu as pltpu

_BF = jnp.bfloat16
_F32 = jnp.float32


def _dot(a, b):
    return jnp.dot(a, b, preferred_element_type=_F32)


def _make_body(SB, T, V):
    TV = T * V

    def _flat_to_3d(flat_bf, r3, C):
        for v in range(V):
            r3[:, v, :] = flat_bf[:, C * v:C * (v + 1)]

    def _to_flat(t3_bf, C):
        return jnp.concatenate([t3_bf[:, v, :] for v in range(V)], axis=1)

    def _temporal64(a3, s, wg0, wg1, w8):
        z3 = jnp.zeros((4, V, 64), _BF)
        yp3 = jnp.concatenate([z3, a3[s * T:(s + 1) * T], z3], axis=0)
        q3 = jnp.concatenate([yp3[i:T + 4 + i] for i in range(4)], axis=2)
        acc = _dot(q3[0:T].reshape(TV, 256), wg0)
        acc = acc + _dot(q3[4:T + 4].reshape(TV, 256), wg1)
        acc = acc + _dot(yp3[8:T + 8].reshape(TV, 64), w8)
        return acc

    def _body(x_ref,
              w0_ref, b0_ref, g00_ref, g01_ref, w80_ref, bt0_ref,
              w1_ref, b1_ref, g10_ref, g11_ref, w81_ref, bt1_ref,
              w2_ref, b2_ref, g2_ref, w82_ref, wr2_ref, bt2_ref,
              o_ref, r3a, r3c):
        x4 = x_ref[...].astype(_BF)
        Cin = x4.shape[1]
        x2d = jnp.concatenate(
            [x4[:, c].reshape(SB * T, V) for c in range(Cin)], axis=1)

        a0 = jnp.maximum(_dot(x2d, w0_ref[...]) + b0_ref[...], 0.0)
        _flat_to_3d(a0.astype(_BF), r3a, 64)
        a3 = r3a[...]
        t0 = [jnp.maximum(
            _temporal64(a3, s, g00_ref[...], g01_ref[...], w80_ref[...])
            + bt0_ref[...], 0.0) for s in range(SB)]

        flat1 = jnp.concatenate(
            [_to_flat(t0[s].astype(_BF).reshape(T, V, 64), 64)
             for s in range(SB)], axis=0)
        a1 = jnp.maximum(_dot(flat1, w1_ref[...]) + b1_ref[...], 0.0)
        _flat_to_3d(a1.astype(_BF), r3a, 64)
        a3 = r3a[...]
        t1 = [jnp.maximum(
            _temporal64(a3, s, g10_ref[...], g11_ref[...], w81_ref[...])
            + t0[s] + bt1_ref[...], 0.0) for s in range(SB)]

        t1b = [t1[s].astype(_BF) for s in range(SB)]
        flat2 = jnp.concatenate(
            [_to_flat(t1b[s].reshape(T, V, 64), 64) for s in range(SB)],
            axis=0)
        a2 = jnp.maximum(_dot(flat2, w2_ref[...]) + b2_ref[...], 0.0)
        _flat_to_3d(a2.astype(_BF), r3c, 32)
        a3c = r3c[...]
        z3 = jnp.zeros((4, V, 32), _BF)
        for s in range(SB):
            yp3 = jnp.concatenate([z3, a3c[s * T:(s + 1) * T], z3], axis=0)
            q3 = jnp.concatenate([yp3[i:T + i] for i in range(8)], axis=2)
            acc = _dot(q3.reshape(TV, 256), g2_ref[...])
            acc = acc + _dot(yp3[8:T + 8].reshape(TV, 32), w82_ref[...])
            acc = acc + _dot(t1b[s], wr2_ref[...])
            acc = acc + bt2_ref[...]
            o_ref[s] = jnp.maximum(acc, 0.0).T.reshape(32, T, V)

    return _body


def _pack_taps(wt, lo, hi):
    n = hi - lo
    return wt[lo:hi].reshape(n * wt.shape[1], wt.shape[2]).astype(_BF)


def _const_spec(shape):
    return pl.BlockSpec(shape, lambda g: tuple(0 for _ in shape))


def kernel(x,
           l0_wbig, l0_bsp, l0_wt, l0_bt,
           l1_wbig, l1_bsp, l1_wt, l1_bt,
           l2_wbig, l2_bsp, l2_wt, l2_bt, l2_wres, l2_bres):
    N, C, T, V = x.shape
    SB = 4
    if N % SB:
        x = jnp.pad(x, ((0, SB - N % SB), (0, 0), (0, 0), (0, 0)))
    Np = x.shape[0]

    w0p = l0_wbig.reshape(V, C, l0_wbig.shape[1]).transpose(1, 0, 2)
    w0p = w0p.reshape(V * C, l0_wbig.shape[1])

    args = (
        x,
        w0p.astype(_BF), l0_bsp,
        _pack_taps(l0_wt, 0, 4), _pack_taps(l0_wt, 4, 8),
        l0_wt[8].astype(_BF), l0_bt,
        l1_wbig.astype(_BF), l1_bsp,
        _pack_taps(l1_wt, 0, 4), _pack_taps(l1_wt, 4, 8),
        l1_wt[8].astype(_BF), l1_bt,
        l2_wbig.astype(_BF), l2_bsp,
        _pack_taps(l2_wt, 0, 8), l2_wt[8].astype(_BF),
        l2_wres.astype(_BF), l2_bt + l2_bres,
    )
    in_specs = [pl.BlockSpec((SB, C, T, V), lambda g: (g, 0, 0, 0))]
    in_specs += [_const_spec(a.shape) for a in args[1:]]

    out = pl.pallas_call(
        _make_body(SB, T, V),
        out_shape=jax.ShapeDtypeStruct((Np, 32, T, V), _F32),
        grid=(Np // SB,),
        in_specs=in_specs,
        out_specs=pl.BlockSpec((SB, 32, T, V), lambda g: (g, 0, 0, 0)),
        scratch_shapes=[pltpu.VMEM((SB * T, V, 64), _BF),
                        pltpu.VMEM((SB * T, V, 32), _BF)],
        compiler_params=pltpu.CompilerParams(
            dimension_semantics=("parallel",),
            vmem_limit_bytes=58 * 1024 * 1024,
        ),
    )(*args)

    return out[:N]


# speedup vs baseline: 1.0065x; 1.0065x over previous
import jax
import jax.numpy as jnp
from jax.experimental import pallas as pl
from jax.experimental.pallas import tpu as pltpu

_BF = jnp.bfloat16
_F32 = jnp.float32

_CP = pltpu.CompilerParams(
    dimension_semantics=("parallel",),
    vmem_limit_bytes=58 * 1024 * 1024,
)


def _dot(a, b):
    return jnp.dot(a, b, preferred_element_type=_F32)


def _const_spec(shape):
    return pl.BlockSpec(shape, lambda g: tuple(0 for _ in shape))


def _flat_to_rows(flat_bf, r3, V, C, SBT):
    for v in range(V):
        r3[:, v, :] = flat_bf[:, C * v:C * (v + 1)]
    return r3[...].reshape(SBT * V, C)


def _temporal64_acc(rows_bf, s, wg0, wg1, w8, TV):
    z = jnp.zeros((128, 64), _BF)
    ys = rows_bf[s * TV:(s + 1) * TV, :]
    yp = jnp.concatenate([z, ys, z], axis=0)
    q = jnp.concatenate(
        [yp[0:TV + 128], yp[32:TV + 160],
         yp[64:TV + 192], yp[96:TV + 224]], axis=1)
    acc = _dot(q[0:TV], wg0)
    acc = acc + _dot(q[128:TV + 128], wg1)
    acc = acc + _dot(yp[256:TV + 256], w8)
    return acc


def _make_body0(SB, T, V):
    TV = T * V

    def _body(x_ref, w0_ref, b0_ref, g0_ref, g1_ref, w8_ref, bt_ref,
              o_ref, r3):
        x4 = x_ref[...].astype(_BF)
        Cin = x4.shape[1]
        x2d = jnp.concatenate(
            [x4[:, c].reshape(SB * T, V) for c in range(Cin)], axis=1)
        a = jnp.maximum(_dot(x2d, w0_ref[...]) + b0_ref[...], 0.0)
        rows = _flat_to_rows(a.astype(_BF), r3, V, 64, SB * T)
        for s in range(SB):
            acc = _temporal64_acc(rows, s, g0_ref[...], g1_ref[...],
                                  w8_ref[...], TV)
            acc = acc + bt_ref[...]
            o_ref[s * TV:(s + 1) * TV, :] = jnp.maximum(acc, 0.0).astype(_BF)

    return _body


def _make_body1(SB, T, V):
    TV = T * V

    def _body(yf_ref, res_ref, w_ref, b_ref, g0_ref, g1_ref, w8_ref, bt_ref,
              o_ref, r3):
        a = jnp.maximum(_dot(yf_ref[...], w_ref[...]) + b_ref[...], 0.0)
        rows = _flat_to_rows(a.astype(_BF), r3, V, 64, SB * T)
        for s in range(SB):
            acc = _temporal64_acc(rows, s, g0_ref[...], g1_ref[...],
                                  w8_ref[...], TV)
            acc = acc + res_ref[s * TV:(s + 1) * TV, :].astype(_F32)
            acc = acc + bt_ref[...]
            o_ref[s * TV:(s + 1) * TV, :] = jnp.maximum(acc, 0.0).astype(_BF)

    return _body


def _make_body2(SB, T, V):
    TV = T * V

    def _body(yf_ref, res_ref, w_ref, b_ref, g_ref, w8_ref, wr_ref, bt_ref,
              o_ref, r3):
        a = jnp.maximum(_dot(yf_ref[...], w_ref[...]) + b_ref[...], 0.0)
        rows = _flat_to_rows(a.astype(_BF), r3, V, 32, SB * T)
        z = jnp.zeros((128, 32), _BF)
        for s in range(SB):
            ys = rows[s * TV:(s + 1) * TV, :]
            yp = jnp.concatenate([z, ys, z], axis=0)
            q = jnp.concatenate([yp[32 * i:32 * i + TV] for i in range(8)],
                                axis=1)
            acc = _dot(q, g_ref[...])
            acc = acc + _dot(yp[256:TV + 256], w8_ref[...])
            acc = acc + _dot(res_ref[s * TV:(s + 1) * TV, :], wr_ref[...])
            acc = acc + bt_ref[...]
            o_ref[s] = jnp.maximum(acc, 0.0).T.reshape(32, T, V)

    return _body


def _pack_taps(wt, lo, hi):
    n = hi - lo
    return wt[lo:hi].reshape(n * wt.shape[1], wt.shape[2]).astype(_BF)


def kernel(x,
           l0_wbig, l0_bsp, l0_wt, l0_bt,
           l1_wbig, l1_bsp, l1_wt, l1_bt,
           l2_wbig, l2_bsp, l2_wt, l2_bt, l2_wres, l2_bres):
    N, C, T, V = x.shape
    SB = 4
    if N % SB:
        x = jnp.pad(x, ((0, SB - N % SB), (0, 0), (0, 0), (0, 0)))
    Np = x.shape[0]
    TV = T * V

    w0p = l0_wbig.reshape(V, C, l0_wbig.shape[1]).transpose(1, 0, 2)
    w0p = w0p.reshape(V * C, l0_wbig.shape[1]).astype(_BF)

    args0 = (x, w0p, l0_bsp, _pack_taps(l0_wt, 0, 4), _pack_taps(l0_wt, 4, 8),
             l0_wt[8].astype(_BF), l0_bt)
    t0 = pl.pallas_call(
        _make_body0(SB, T, V),
        out_shape=jax.ShapeDtypeStruct((Np * TV, 64), _BF),
        grid=(Np // SB,),
        in_specs=[pl.BlockSpec((SB, C, T, V), lambda g: (g, 0, 0, 0))]
        + [_const_spec(a.shape) for a in args0[1:]],
        out_specs=pl.BlockSpec((SB * TV, 64), lambda g: (g, 0)),
        scratch_shapes=[pltpu.VMEM((SB * T, V, 64), _BF)],
        compiler_params=_CP,
    )(*args0)

    t0f = t0.reshape(Np * T, V * 64)
    args1 = (t0f, t0, l1_wbig.astype(_BF), l1_bsp,
             _pack_taps(l1_wt, 0, 4), _pack_taps(l1_wt, 4, 8),
             l1_wt[8].astype(_BF), l1_bt)
    t1 = pl.pallas_call(
        _make_body1(SB, T, V),
        out_shape=jax.ShapeDtypeStruct((Np * TV, 64), _BF),
        grid=(Np // SB,),
        in_specs=[pl.BlockSpec((SB * T, V * 64), lambda g: (g, 0)),
                  pl.BlockSpec((SB * TV, 64), lambda g: (g, 0))]
        + [_const_spec(a.shape) for a in args1[2:]],
        out_specs=pl.BlockSpec((SB * TV, 64), lambda g: (g, 0)),
        scratch_shapes=[pltpu.VMEM((SB * T, V, 64), _BF)],
        compiler_params=_CP,
    )(*args1)

    t1f = t1.reshape(Np * T, V * 64)
    args2 = (t1f, t1, l2_wbig.astype(_BF), l2_bsp,
             _pack_taps(l2_wt, 0, 8), l2_wt[8].astype(_BF),
             l2_wres.astype(_BF), l2_bt + l2_bres)
    out = pl.pallas_call(
        _make_body2(SB, T, V),
        out_shape=jax.ShapeDtypeStruct((Np, 32, T, V), _F32),
        grid=(Np // SB,),
        in_specs=[pl.BlockSpec((SB * T, V * 64), lambda g: (g, 0)),
                  pl.BlockSpec((SB * TV, 64), lambda g: (g, 0))]
        + [_const_spec(a.shape) for a in args2[2:]],
        out_specs=pl.BlockSpec((SB, 32, T, V), lambda g: (g, 0, 0, 0)),
        scratch_shapes=[pltpu.VMEM((SB * T, V, 32), _BF)],
        compiler_params=_CP,
    )(*args2)

    return out[:N]


# speedup vs baseline: 1.1421x; 1.1346x over previous
import jax
import jax.numpy as jnp
from jax.experimental import pallas as pl
from jax.experimental.pallas import tpu as pltpu

_BF = jnp.bfloat16
_F32 = jnp.float32

_CP = pltpu.CompilerParams(
    dimension_semantics=("parallel",),
    vmem_limit_bytes=58 * 1024 * 1024,
)


def _dot(a, b):
    return jnp.dot(a, b, preferred_element_type=_F32)


def _const_spec(shape):
    return pl.BlockSpec(shape, lambda g: tuple(0 for _ in shape))


def _spatial_body(x_ref, w_ref, b_ref, o_ref):
    y = _dot(x_ref[...], w_ref[...])
    o_ref[...] = jnp.maximum(y + b_ref[...], 0.0).astype(o_ref.dtype)


def _spatial(x2d, w, b, rows):
    M, K = x2d.shape
    Nout = w.shape[1]
    return pl.pallas_call(
        _spatial_body,
        out_shape=jax.ShapeDtypeStruct((M, Nout), _BF),
        grid=(M // rows,),
        in_specs=[pl.BlockSpec((rows, K), lambda g: (g, 0)),
                  _const_spec((K, Nout)), _const_spec((1, Nout))],
        out_specs=pl.BlockSpec((rows, Nout), lambda g: (g, 0)),
        compiler_params=_CP,
    )(x2d, w, b)


def _make_spatial0_body(SB, T, V):
    def _body(x_ref, w_ref, b_ref, o_ref):
        x4 = x_ref[...].astype(_BF)
        Cin = x4.shape[1]
        x2d = jnp.concatenate(
            [x4[:, c].reshape(SB * T, V) for c in range(Cin)], axis=1)
        y = _dot(x2d, w_ref[...])
        o_ref[...] = jnp.maximum(y + b_ref[...], 0.0).astype(o_ref.dtype)

    return _body


def _spatial0(x, w, b, *, SB, T, V):
    Np, C = x.shape[0], x.shape[1]
    Nout = w.shape[1]
    return pl.pallas_call(
        _make_spatial0_body(SB, T, V),
        out_shape=jax.ShapeDtypeStruct((Np * T, Nout), _BF),
        grid=(Np // SB,),
        in_specs=[pl.BlockSpec((SB, C, T, V), lambda g: (g, 0, 0, 0)),
                  _const_spec((V * C, Nout)), _const_spec((1, Nout))],
        out_specs=pl.BlockSpec((SB * T, Nout), lambda g: (g, 0)),
        compiler_params=_CP,
    )(x, w, b)


def _temporal64_acc(y_ref, s, wg0, wg1, w8, TV):
    z = jnp.zeros((128, 64), _BF)
    ys = y_ref[s * TV:(s + 1) * TV, :]
    yp = jnp.concatenate([z, ys, z], axis=0)
    q = jnp.concatenate(
        [yp[0:TV + 128], yp[32:TV + 160],
         yp[64:TV + 192], yp[96:TV + 224]], axis=1)
    acc = _dot(q[0:TV], wg0)
    acc = acc + _dot(q[128:TV + 128], wg1)
    acc = acc + _dot(yp[256:TV + 256], w8)
    return acc


def _make_temporal64_body(res, *, SB, TV):
    def _body(*refs):
        if res:
            y_ref, r_ref, g0_ref, g1_ref, w8_ref, bt_ref, o_ref = refs
        else:
            y_ref, g0_ref, g1_ref, w8_ref, bt_ref, o_ref = refs
        for s in range(SB):
            acc = _temporal64_acc(y_ref, s, g0_ref[...], g1_ref[...],
                                  w8_ref[...], TV)
            if res:
                acc = acc + r_ref[s * TV:(s + 1) * TV, :].astype(_F32)
            acc = acc + bt_ref[...]
            o_ref[s * TV:(s + 1) * TV, :] = jnp.maximum(acc, 0.0).astype(_BF)

    return _body


def _temporal64(y_rows, res_rows, wg0, wg1, w8, bt, *, SB, TV):
    M = y_rows.shape[0]
    rows = SB * TV
    if res_rows is None:
        args = (y_rows, wg0, wg1, w8, bt)
        in_specs = [pl.BlockSpec((rows, 64), lambda g: (g, 0))]
    else:
        args = (y_rows, res_rows, wg0, wg1, w8, bt)
        in_specs = [pl.BlockSpec((rows, 64), lambda g: (g, 0)),
                    pl.BlockSpec((rows, 64), lambda g: (g, 0))]
    in_specs += [_const_spec((256, 64)), _const_spec((256, 64)),
                 _const_spec((64, 64)), _const_spec((1, 64))]
    return pl.pallas_call(
        _make_temporal64_body(res_rows is not None, SB=SB, TV=TV),
        out_shape=jax.ShapeDtypeStruct((M, 64), _BF),
        grid=(M // rows,),
        in_specs=in_specs,
        out_specs=pl.BlockSpec((rows, 64), lambda g: (g, 0)),
        compiler_params=_CP,
    )(*args)


def _make_temporal32_body(*, SB, T, V):
    TV = T * V

    def _body(y_ref, r_ref, g_ref, w8_ref, wr_ref, b_ref, o_ref):
        z = jnp.zeros((128, 32), _BF)
        for s in range(SB):
            ys = y_ref[s * TV:(s + 1) * TV, :]
            yp = jnp.concatenate([z, ys, z], axis=0)
            q = jnp.concatenate([yp[32 * i:32 * i + TV] for i in range(8)],
                                axis=1)
            acc = _dot(q, g_ref[...])
            acc = acc + _dot(yp[256:TV + 256], w8_ref[...])
            acc = acc + _dot(r_ref[s * TV:(s + 1) * TV, :], wr_ref[...])
            acc = acc + b_ref[...]
            o_ref[s] = jnp.maximum(acc, 0.0).T.reshape(32, T, V)

    return _body


def _temporal32(y_rows, res_rows, wg, w8, wr, b, *, SB, T, V):
    TV = T * V
    Np = y_rows.shape[0] // TV
    return pl.pallas_call(
        _make_temporal32_body(SB=SB, T=T, V=V),
        out_shape=jax.ShapeDtypeStruct((Np, 32, T, V), _F32),
        grid=(Np // SB,),
        in_specs=[pl.BlockSpec((SB * TV, 32), lambda g: (g, 0)),
                  pl.BlockSpec((SB * TV, 64), lambda g: (g, 0)),
                  _const_spec((256, 32)), _const_spec((32, 32)),
                  _const_spec((64, 32)), _const_spec((1, 32))],
        out_specs=pl.BlockSpec((SB, 32, T, V), lambda g: (g, 0, 0, 0)),
        compiler_params=_CP,
    )(y_rows, res_rows, wg, w8, wr, b)


def _pack_taps(wt, lo, hi):
    n = hi - lo
    return wt[lo:hi].reshape(n * wt.shape[1], wt.shape[2]).astype(_BF)


def kernel(x,
           l0_wbig, l0_bsp, l0_wt, l0_bt,
           l1_wbig, l1_bsp, l1_wt, l1_bt,
           l2_wbig, l2_bsp, l2_wt, l2_bt, l2_wres, l2_bres):
    N, C, T, V = x.shape
    SB = 8
    if N % SB:
        x = jnp.pad(x, ((0, SB - N % SB), (0, 0), (0, 0), (0, 0)))
    Np = x.shape[0]
    TV = T * V

    w0p = l0_wbig.reshape(V, C, l0_wbig.shape[1]).transpose(1, 0, 2)
    w0p = w0p.reshape(V * C, l0_wbig.shape[1]).astype(_BF)

    y0 = _spatial0(x, w0p, l0_bsp, SB=SB, T=T, V=V)
    t0 = _temporal64(y0.reshape(Np * TV, 64), None,
                     _pack_taps(l0_wt, 0, 4), _pack_taps(l0_wt, 4, 8),
                     l0_wt[8].astype(_BF), l0_bt, SB=SB, TV=TV)

    y1 = _spatial(t0.reshape(Np * T, V * 64), l1_wbig.astype(_BF), l1_bsp,
                  SB * T)
    t1 = _temporal64(y1.reshape(Np * TV, 64), t0,
                     _pack_taps(l1_wt, 0, 4), _pack_taps(l1_wt, 4, 8),
                     l1_wt[8].astype(_BF), l1_bt, SB=SB, TV=TV)

    y2 = _spatial(t1.reshape(Np * T, V * 64), l2_wbig.astype(_BF), l2_bsp,
                  SB * T)
    out = _temporal32(y2.reshape(Np * TV, 32), t1,
                      _pack_taps(l2_wt, 0, 8), l2_wt[8].astype(_BF),
                      l2_wres.astype(_BF), l2_bt + l2_bres, SB=SB, T=T, V=V)

    return out[:N]
